# Initial kernel scaffold; baseline (speedup 1.0000x reference)
import jax
import jax.numpy as jnp
from jax.experimental import pallas as pl


def kernel(id_batch, neigh_0, neigh_1, neigh_2, emb_0, emb_1, emb_2, Wih, Whh, bih, bhh, sem):
    raise NotImplementedError("write your pallas kernel here")


# speedup vs baseline: 8.6752x; 8.6752x over previous
import functools

import jax
import jax.numpy as jnp
from jax import lax
from jax.experimental import pallas as pl
from jax.experimental.pallas import tpu as pltpu
from jax.experimental.pallas import tpu_sc as plsc

EMBED_D = 128
N_LAYERS = 2
N_TYPES = 3
S = 10
B = 8192

NW = 32
CH = 128
BB = 256


def _sc_gather(emb_0, emb_1, emb_2, sidx, i0, i1, i2):
    info = plsc.get_sparse_core_info()
    nc = info.num_cores
    nsc = (B // NW) // CH
    nnc = (B * S // NW) // CH
    mesh = plsc.VectorSubcoreMesh(core_axis_name="c", subcore_axis_name="s")

    @functools.partial(
        pl.kernel,
        mesh=mesh,
        out_type=(
            jax.ShapeDtypeStruct((B, EMBED_D), jnp.float32),
            jax.ShapeDtypeStruct((B * S, EMBED_D), jnp.float32),
            jax.ShapeDtypeStruct((B * S, EMBED_D), jnp.float32),
            jax.ShapeDtypeStruct((B * S, EMBED_D), jnp.float32),
        ),
        scratch_types=[
            pltpu.VMEM((nsc, CH), jnp.int32),
            pltpu.VMEM((nnc, CH), jnp.int32),
            pltpu.VMEM((CH, EMBED_D), jnp.float32),
            pltpu.VMEM((CH, EMBED_D), jnp.float32),
            pltpu.SemaphoreType.DMA,
            pltpu.SemaphoreType.DMA,
        ],
    )
    def k(e0, e1, e2, sidx_h, i0_h, i1_h, i2_h, out_s, o0, o1, o2,
          sidx_v, idx_v, rows_a, rows_b, sem_a, sem_b):
        wid = lax.axis_index("s") * nc + lax.axis_index("c")
        rows = (rows_a, rows_b)
        sems = (sem_a, sem_b)

        pltpu.sync_copy(sidx_h.at[wid], sidx_v)
        base = wid * (B // NW)
        for j in range(nsc):
            r, sm = rows[j % 2], sems[j % 2]
            pltpu.async_copy(e0.at[sidx_v.at[j]], r, sm).wait()
            pltpu.sync_copy(r, out_s.at[pl.ds(base + j * CH, CH)])

        nb_base = wid * (B * S // NW)
        for tbl, idx_h, out in ((e0, i0_h, o0), (e1, i1_h, o1), (e2, i2_h, o2)):
            pltpu.sync_copy(idx_h.at[wid], idx_v)
            cp = pltpu.async_copy(tbl.at[idx_v.at[0]], rows[0], sems[0])
            for j in range(nnc):
                nxt = None
                if j + 1 < nnc:
                    nxt = pltpu.async_copy(
                        tbl.at[idx_v.at[j + 1]], rows[(j + 1) % 2], sems[(j + 1) % 2])
                cp.wait()
                pltpu.sync_copy(rows[j % 2], out.at[pl.ds(nb_base + j * CH, CH)])
                cp = nxt

    return k(emb_0, emb_1, emb_2, sidx, i0, i1, i2)


def _tc_body(cur_ref, x0, x1, x2, wih, whh, bih, bhh, sem, out_ref):
    xs = (x0, x1, x2)
    dn = (((1,), (1,)), ((), ()))

    xps = []
    for l in range(N_LAYERS):
        for t in range(N_TYPES):
            x = xs[t][:].reshape(S * BB, EMBED_D)
            xp = lax.dot_general(x, wih[l, t], dn,
                                 preferred_element_type=jnp.float32)
            xps.append(xp + bih[l, t])

    h = [None] * (N_LAYERS * N_TYPES)
    acc = [None] * (N_LAYERS * N_TYPES)
    for s in range(S):
        for k in range(N_LAYERS * N_TYPES):
            l, t = divmod(k, N_TYPES)
            xp_s = xps[k][s * BB:(s + 1) * BB, :]
            if s == 0:
                hn = jnp.tanh(xp_s + bhh[l, t])
                acc[k] = hn
            else:
                hn = jnp.tanh(
                    xp_s + bhh[l, t]
                    + lax.dot_general(h[k], whh[l, t], dn,
                                      preferred_element_type=jnp.float32))
                acc[k] = acc[k] + hn
            h[k] = hn
    aggs = [a * (1.0 / S) for a in acc]

    cur = cur_ref[:]
    for l in range(N_LAYERS):
        s1 = sem[l, 0]
        s2 = sem[l, 1]
        cs1 = jnp.sum(cur * s1, axis=1, keepdims=True)
        lg = [cs1 + jnp.sum(cur * s2, axis=1, keepdims=True)]
        for t in range(N_TYPES):
            lg.append(cs1 + jnp.sum(aggs[l * 3 + t] * s2, axis=1, keepdims=True))
        m = jnp.maximum(jnp.maximum(lg[0], lg[1]), jnp.maximum(lg[2], lg[3]))
        e = [jnp.exp(v - m) for v in lg]
        den = e[0] + e[1] + e[2] + e[3]
        mix = (e[0] * cur + e[1] * aggs[l * 3]
               + e[2] * aggs[l * 3 + 1] + e[3] * aggs[l * 3 + 2]) / den
        cur = jnp.where(mix > 0, mix, 0.01 * mix)
    out_ref[:] = cur


def _tc_compute(cur0, g0, g1, g2, Wih, Whh, bih, bhh, sem):
    grid = (B // BB,)
    xspec = pl.BlockSpec((S, BB, EMBED_D), lambda i: (0, i, 0))
    wspec = pl.BlockSpec((N_LAYERS, N_TYPES, EMBED_D, EMBED_D),
                         lambda i: (0, 0, 0, 0))
    bspec = pl.BlockSpec((N_LAYERS, N_TYPES, 1, EMBED_D), lambda i: (0, 0, 0, 0))
    sspec = pl.BlockSpec((N_LAYERS, 2, 1, EMBED_D), lambda i: (0, 0, 0, 0))
    return pl.pallas_call(
        _tc_body,
        grid=grid,
        in_specs=[
            pl.BlockSpec((BB, EMBED_D), lambda i: (i, 0)),
            xspec, xspec, xspec,
            wspec, wspec, bspec, bspec, sspec,
        ],
        out_specs=pl.BlockSpec((BB, EMBED_D), lambda i: (i, 0)),
        out_shape=jax.ShapeDtypeStruct((B, EMBED_D), jnp.float32),
    )(cur0, g0, g1, g2, Wih, Whh,
      bih.reshape(N_LAYERS, N_TYPES, 1, EMBED_D),
      bhh.reshape(N_LAYERS, N_TYPES, 1, EMBED_D),
      sem.reshape(N_LAYERS, 2, 1, EMBED_D))


def kernel(id_batch, neigh_0, neigh_1, neigh_2, emb_0, emb_1, emb_2,
           Wih, Whh, bih, bhh, sem):
    sidx = id_batch.astype(jnp.int32).reshape(NW, (B // NW) // CH, CH)
    idxs = [n.astype(jnp.int32).T.reshape(NW, (B * S // NW) // CH, CH)
            for n in (neigh_0, neigh_1, neigh_2)]
    cur0, g0, g1, g2 = _sc_gather(emb_0, emb_1, emb_2, sidx, *idxs)
    g0 = g0.reshape(S, B, EMBED_D)
    g1 = g1.reshape(S, B, EMBED_D)
    g2 = g2.reshape(S, B, EMBED_D)
    return _tc_compute(cur0, g0, g1, g2, Wih, Whh, bih, bhh, sem)
